# Initial kernel scaffold; baseline (speedup 1.0000x reference)
import jax
import jax.numpy as jnp
from jax.experimental import pallas as pl


def kernel(x, A0, A1, A2, A3, B0, B1, B2, B3):
    raise NotImplementedError("write your pallas kernel here")


# speedup vs baseline: 1.0951x; 1.0951x over previous
import jax
import jax.numpy as jnp
from jax.experimental import pallas as pl
from jax.experimental.pallas import tpu as pltpu

B = 1024
I = 16
F = 512
K = 512
N_LAYERS = 4
BB = 256


def _body(x_ref, a0, a1, a2, a3, b0, b1, b2, b3,
          l0, l1, l2, l3, n0, n1, n2, n3,
          a_sc, b_sc):
    bb = pl.program_id(1)
    a_refs = (a0, a1, a2, a3)
    b_refs = (b0, b1, b2, b3)

    @pl.when(bb == 0)
    def _prepare():
        for l in range(N_LAYERS):
            a = a_refs[l][0]
            inv = jax.lax.rsqrt(jnp.sum(a * a, axis=0, keepdims=True))
            a_sc[l] = (a * inv).astype(jnp.bfloat16)
            b_sc[l] = b_refs[l][0].astype(jnp.bfloat16)

    inner_outs = (n0, n1, n2, n3)
    layer_outs = (l0, l1, l2, l3)
    x = x_ref[...]
    for l in range(N_LAYERS):
        inner = jnp.dot(x.astype(jnp.bfloat16), a_sc[l],
                        preferred_element_type=jnp.float32)
        inner_outs[l][...] = inner
        x = jnp.dot(inner.astype(jnp.bfloat16), b_sc[l],
                    preferred_element_type=jnp.float32)
        layer_outs[l][...] = x


def kernel(x, A0, A1, A2, A3, B0, B1, B2, B3):
    x2 = x.reshape(B, I * F)
    w_spec = pl.BlockSpec((1, F, K), lambda i, b: (i, 0, 0))
    act_spec = pl.BlockSpec((BB, F), lambda i, b: (b, i))
    out_shape = [jax.ShapeDtypeStruct((B, I * F), jnp.float32)
                 for _ in range(2 * N_LAYERS)]
    outs = pl.pallas_call(
        _body,
        grid=(I, B // BB),
        in_specs=[act_spec] + [w_spec] * (2 * N_LAYERS),
        out_specs=[act_spec] * (2 * N_LAYERS),
        out_shape=out_shape,
        scratch_shapes=[
            pltpu.VMEM((N_LAYERS, F, K), jnp.bfloat16),
            pltpu.VMEM((N_LAYERS, K, F), jnp.bfloat16),
        ],
        compiler_params=pltpu.CompilerParams(
            dimension_semantics=("arbitrary", "arbitrary"),
        ),
    )(x2, A0, A1, A2, A3, B0, B1, B2, B3)
    l0, l1, l2, l3, n0, n1, n2, n3 = [o.reshape(B, I, F) for o in outs]
    return (l3, l0, l1, l2, l3, n0, n1, n2, n3)


# speedup vs baseline: 1.7229x; 1.5732x over previous
import jax
import jax.numpy as jnp
from jax.experimental import pallas as pl
from jax.experimental.pallas import tpu as pltpu

B = 1024
I = 16
F = 512
K = 512
N_LAYERS = 4
BB = 128
OCT = 8


def _prep_weights(a_ref, b_ref, a_sc, b_sc):
    a = a_ref[...]
    inv = jax.lax.rsqrt(jnp.sum(a * a, axis=1, keepdims=True))
    a_sc[...] = (a * inv).astype(jnp.bfloat16)
    b_sc[...] = b_ref[...].astype(jnp.bfloat16)


def _layer_body(x_ref, a_ref, b_ref, inner_ref, xnew_ref, xflat_ref,
                a_sc, b_sc):
    bb = pl.program_id(1)
    pl.when(bb == 0)(lambda: _prep_weights(a_ref, b_ref, a_sc, b_sc))
    for i in range(OCT):
        xi = x_ref[:, i * F:(i + 1) * F]
        inner = jnp.dot(xi, a_sc[i], preferred_element_type=jnp.float32)
        inner_ref[:, i, :] = inner
        xn = jnp.dot(inner.astype(jnp.bfloat16), b_sc[i],
                     preferred_element_type=jnp.float32)
        xnew_ref[:, i, :] = xn
        xflat_ref[:, i * F:(i + 1) * F] = xn.astype(jnp.bfloat16)


def _layer_body_last(x_ref, a_ref, b_ref, inner_ref, xnew_ref, xdup_ref,
                     a_sc, b_sc):
    bb = pl.program_id(1)
    pl.when(bb == 0)(lambda: _prep_weights(a_ref, b_ref, a_sc, b_sc))
    for i in range(OCT):
        xi = x_ref[:, i * F:(i + 1) * F]
        inner = jnp.dot(xi, a_sc[i], preferred_element_type=jnp.float32)
        inner_ref[:, i, :] = inner
        xn = jnp.dot(inner.astype(jnp.bfloat16), b_sc[i],
                     preferred_element_type=jnp.float32)
        xnew_ref[:, i, :] = xn
        xdup_ref[:, i, :] = xn


def _layer(x_flat, A, Bw, last):
    flat_spec = pl.BlockSpec((BB, OCT * F), lambda o, bb: (bb, o))
    can_spec = pl.BlockSpec((BB, OCT, F), lambda o, bb: (bb, o, 0))
    w_spec = pl.BlockSpec((OCT, F, K), lambda o, bb: (o, 0, 0))
    can_shape = jax.ShapeDtypeStruct((B, I, F), jnp.float32)
    flat_shape = jax.ShapeDtypeStruct((B, I * F), jnp.bfloat16)
    out_specs = [can_spec, can_spec, can_spec if last else flat_spec]
    out_shape = [can_shape, can_shape, can_shape if last else flat_shape]
    return pl.pallas_call(
        _layer_body_last if last else _layer_body,
        grid=(I // OCT, B // BB),
        in_specs=[flat_spec, w_spec, w_spec],
        out_specs=out_specs,
        out_shape=out_shape,
        scratch_shapes=[
            pltpu.VMEM((OCT, F, K), jnp.bfloat16),
            pltpu.VMEM((OCT, K, F), jnp.bfloat16),
        ],
        compiler_params=pltpu.CompilerParams(
            dimension_semantics=("arbitrary", "arbitrary"),
        ),
    )(x_flat, A, Bw)


def kernel(x, A0, A1, A2, A3, B0, B1, B2, B3):
    x_flat = x.reshape(B, I * F).astype(jnp.bfloat16)
    n0, l0, f0 = _layer(x_flat, A0, B0, last=False)
    n1, l1, f1 = _layer(f0, A1, B1, last=False)
    n2, l2, f2 = _layer(f1, A2, B2, last=False)
    n3, l3, xfin = _layer(f2, A3, B3, last=True)
    return (xfin, l0, l1, l2, l3, n0, n1, n2, n3)
